# baseline (speedup 1.0000x reference)
import jax
import jax.numpy as jnp
from jax.experimental import pallas as pl

CELL = 7
BC = 2
NC = 20
NCH = NC + BC * 5
NCELL = CELL * CELL


def _decode_kernel(x_ref, boxes_ref, scores_ref, idxs_ref):
    xb = x_ref[...]
    cls = xb[:, -NC:, :]

    lane = jax.lax.broadcasted_iota(jnp.int32, (1, NCELL), 1)
    gx = (lane % CELL).astype(jnp.float32)
    gy = (lane // CELL).astype(jnp.float32)

    for i in range(BC):
        conf = xb[:, i * 5 + 4, :]
        det_cls = cls * conf[:, None, :]
        idxs_ref[:, i, :] = jnp.argmax(det_cls, axis=1).astype(jnp.float32)
        scores_ref[:, i, :] = jnp.max(det_cls, axis=1)

        cx = (xb[:, i * 5 + 0, :] + gx) / CELL
        cy = (xb[:, i * 5 + 1, :] + gy) / CELL
        hw = xb[:, i * 5 + 2, :] * 0.5
        hh = xb[:, i * 5 + 3, :] * 0.5
        boxes_ref[:, i, 0, :] = cx - hw
        boxes_ref[:, i, 1, :] = cy - hh
        boxes_ref[:, i, 2, :] = cx + hw
        boxes_ref[:, i, 3, :] = cy + hh


def kernel(x, block_b: int = 512, interpret: bool = False):
    B = x.shape[0]
    xf = x.reshape(B, NCH, NCELL)
    grid = (B // block_b,)
    boxes_t, scores_t, idxs_t = pl.pallas_call(
        _decode_kernel,
        grid=grid,
        in_specs=[pl.BlockSpec((block_b, NCH, NCELL), lambda b: (b, 0, 0))],
        out_specs=[
            pl.BlockSpec((block_b, BC, 4, NCELL), lambda b: (b, 0, 0, 0)),
            pl.BlockSpec((block_b, BC, NCELL), lambda b: (b, 0, 0)),
            pl.BlockSpec((block_b, BC, NCELL), lambda b: (b, 0, 0)),
        ],
        out_shape=[
            jax.ShapeDtypeStruct((B, BC, 4, NCELL), x.dtype),
            jax.ShapeDtypeStruct((B, BC, NCELL), x.dtype),
            jax.ShapeDtypeStruct((B, BC, NCELL), x.dtype),
        ],
        interpret=interpret,
    )(xf)
    boxes = boxes_t.transpose(0, 1, 3, 2).reshape(B, BC * NCELL, 4)
    scores = scores_t.reshape(B, BC * NCELL)
    idxs = idxs_t.reshape(B, BC * NCELL)
    return boxes, scores, idxs
